# Initial kernel scaffold; baseline (speedup 1.0000x reference)
import jax
import jax.numpy as jnp
from jax.experimental import pallas as pl


def kernel(verts, emb, W_pos, b_pos, W0_0, b0_0, W1_0, b1_0, Ws_0, W0_1, b0_1, W1_1, b1_1, Ws_1, W0_2, b0_2, W1_2, b1_2, Ws_2, W_c, b_c):
    raise NotImplementedError("write your pallas kernel here")


# speedup vs baseline: 1.9244x; 1.9244x over previous
import functools

import jax
import jax.numpy as jnp
from jax import lax
from jax.experimental import pallas as pl
from jax.experimental.pallas import tpu as pltpu

_B, _T = 2, 4
_BT = _B * _T
_N = 10475
_F = 128
_RES = 128
_R2 = _RES * _RES
_TILE = 512
_NT = 21
_NP = _TILE * _NT
_UN = _N // 8
_TAIL = _N - 8 * _UN

_NEG = -3.4e38
_ZC = 128


def _resblock(x, w0, b0, w1, b1, ws):
    h = jnp.dot(jnp.maximum(x, 0.0), w0, preferred_element_type=jnp.float32) + b0
    dx = jnp.dot(jnp.maximum(h, 0.0), w1, preferred_element_type=jnp.float32) + b1
    return jnp.dot(x, ws, preferred_element_type=jnp.float32) + dx


def _dense1_body(verts_ref, emb_ref, w3_ref, we_ref, bp_ref,
                 w0_ref, b0_ref, w1_ref, b1_ref, ws_ref, out_ref):
    v = verts_ref[0]
    e = emb_ref[...]
    x = (jnp.dot(v, w3_ref[...], preferred_element_type=jnp.float32)
         + jnp.dot(e, we_ref[...], preferred_element_type=jnp.float32)
         + bp_ref[...])
    out_ref[0] = _resblock(x, w0_ref[...], b0_ref[...], w1_ref[...],
                           b1_ref[...], ws_ref[...])


def _dense1(verts_p, emb_p, w3, we, bp, w0, b0, w1, b1, ws):
    full = lambda shape: pl.BlockSpec(shape, lambda b, t: (0,) * len(shape))
    return pl.pallas_call(
        _dense1_body,
        grid=(_BT, _NT),
        in_specs=[
            pl.BlockSpec((1, _TILE, 8), lambda b, t: (b, t, 0)),
            pl.BlockSpec((_TILE, _F), lambda b, t: (t, 0)),
            full((8, 2 * _F)),
            full((_F, 2 * _F)),
            full((1, 2 * _F)),
            full((2 * _F, _F)),
            full((1, _F)),
            full((_F, _F)),
            full((1, _F)),
            full((2 * _F, _F)),
        ],
        out_specs=pl.BlockSpec((1, _TILE, _F), lambda b, t: (b, t, 0)),
        out_shape=jax.ShapeDtypeStruct((_BT, _NP, _F), jnp.float32),
    )(verts_p, emb_p, w3, we, bp, w0, b0, w1, b1, ws)


def _dense23_body(net_ref, pooled_ref, w0_ref, b0_ref, w1_ref, b1_ref,
                  ws_ref, wc_ref, bc_ref, out_ref, *, project):
    x = jnp.concatenate([net_ref[0], pooled_ref[0]], axis=-1)
    y = _resblock(x, w0_ref[...], b0_ref[...], w1_ref[...], b1_ref[...],
                  ws_ref[...])
    if project:
        y = jnp.dot(y, wc_ref[...], preferred_element_type=jnp.float32) + bc_ref[...]
    out_ref[0] = y


def _dense23(net, pooled, w0, b0, w1, b1, ws, wc, bc, project):
    full = lambda shape: pl.BlockSpec(shape, lambda b, t: (0,) * len(shape))
    return pl.pallas_call(
        functools.partial(_dense23_body, project=project),
        grid=(_BT, _NT),
        in_specs=[
            pl.BlockSpec((1, _TILE, _F), lambda b, t: (b, t, 0)),
            pl.BlockSpec((1, _TILE, _F), lambda b, t: (b, t, 0)),
            full((2 * _F, _F)),
            full((1, _F)),
            full((_F, _F)),
            full((1, _F)),
            full((2 * _F, _F)),
            full((_F, _F)),
            full((1, _F)),
        ],
        out_specs=pl.BlockSpec((1, _TILE, _F), lambda b, t: (b, t, 0)),
        out_shape=jax.ShapeDtypeStruct((_BT, _NP, _F), jnp.float32),
    )(net, pooled, w0, b0, w1, b1, ws, wc, bc)


def _pool_body(idx_ref, net_ref, out_ref, bins_ref):
    p = pl.program_id(1)

    def init_body(r, _):
        bins_ref[pl.ds(r * 128, 128), :] = jnp.full((128, _F), _NEG, jnp.float32)
        return 0
    lax.fori_loop(0, _R2 // 128, init_body, 0)

    @pl.when(p == 0)
    def _():
        def zero_body(r, _):
            out_ref[0, pl.ds(r * _ZC, _ZC), :] = jnp.zeros((_ZC, _F), jnp.float32)
            return 0
        lax.fori_loop(0, _NP // _ZC, zero_body, 0)

    def smax_body(k, _):
        v0 = k * 8
        rows = net_ref[0, pl.ds(v0, 8), :]
        for j in range(8):
            i = idx_ref[0, 0, v0 + j]
            bins_ref[pl.ds(i, 1), :] = jnp.maximum(bins_ref[pl.ds(i, 1), :],
                                                   rows[j:j + 1, :])
        return 0
    lax.fori_loop(0, _UN, smax_body, 0)
    for j in range(_TAIL):
        v = 8 * _UN + j
        i = idx_ref[0, 0, v]
        bins_ref[pl.ds(i, 1), :] = jnp.maximum(bins_ref[pl.ds(i, 1), :],
                                               net_ref[0, pl.ds(v, 1), :])

    def gath_body(k, _):
        v0 = k * 8
        acc = out_ref[0, pl.ds(v0, 8), :]
        parts = []
        for j in range(8):
            i = idx_ref[0, 0, v0 + j]
            parts.append(bins_ref[pl.ds(i, 1), :])
        out_ref[0, pl.ds(v0, 8), :] = acc + jnp.concatenate(parts, axis=0)
        return 0
    lax.fori_loop(0, _UN, gath_body, 0)
    for j in range(_TAIL):
        v = 8 * _UN + j
        i = idx_ref[0, 0, v]
        out_ref[0, pl.ds(v, 1), :] = out_ref[0, pl.ds(v, 1), :] + bins_ref[pl.ds(i, 1), :]


def _pool(net, idx3):
    return pl.pallas_call(
        _pool_body,
        grid=(_BT, 3),
        in_specs=[
            pl.BlockSpec((1, 1, _NP), lambda b, p: (b * 3 + p, 0, 0),
                         memory_space=pltpu.SMEM),
            pl.BlockSpec((1, _NP, _F), lambda b, p: (b, 0, 0)),
        ],
        out_specs=pl.BlockSpec((1, _NP, _F), lambda b, p: (b, 0, 0)),
        out_shape=jax.ShapeDtypeStruct((_BT, _NP, _F), jnp.float32),
        scratch_shapes=[pltpu.VMEM((_R2, _F), jnp.float32)],
    )(idx3, net)


def _mean_body(idx_ref, c_ref, out_ref, cnt_ref):
    def init_body(r, _):
        sl = pl.ds(r * 128, 128)
        out_ref[0, 0, sl, :] = jnp.zeros((128, _F), jnp.float32)
        cnt_ref[sl, :] = jnp.zeros((128, _F), jnp.float32)
        return 0
    lax.fori_loop(0, _R2 // 128, init_body, 0)

    one = jnp.ones((1, _F), jnp.float32)

    def add_body(k, _):
        v0 = k * 8
        rows = c_ref[0, pl.ds(v0, 8), :]
        for j in range(8):
            i = idx_ref[0, 0, v0 + j]
            out_ref[0, 0, pl.ds(i, 1), :] = out_ref[0, 0, pl.ds(i, 1), :] + rows[j:j + 1, :]
            cnt_ref[pl.ds(i, 1), :] = cnt_ref[pl.ds(i, 1), :] + one
        return 0
    lax.fori_loop(0, _UN, add_body, 0)
    for j in range(_TAIL):
        v = 8 * _UN + j
        i = idx_ref[0, 0, v]
        out_ref[0, 0, pl.ds(i, 1), :] = out_ref[0, 0, pl.ds(i, 1), :] + c_ref[0, pl.ds(v, 1), :]
        cnt_ref[pl.ds(i, 1), :] = cnt_ref[pl.ds(i, 1), :] + one

    def div_body(r, _):
        sl = pl.ds(r * 128, 128)
        out_ref[0, 0, sl, :] = out_ref[0, 0, sl, :] / jnp.maximum(cnt_ref[sl, :], 1.0)
        return 0
    lax.fori_loop(0, _R2 // 128, div_body, 0)


def _mean(c, idx3):
    return pl.pallas_call(
        _mean_body,
        grid=(_BT, 3),
        in_specs=[
            pl.BlockSpec((1, 1, _NP), lambda b, p: (b * 3 + p, 0, 0),
                         memory_space=pltpu.SMEM),
            pl.BlockSpec((1, _NP, _F), lambda b, p: (b, 0, 0)),
        ],
        out_specs=pl.BlockSpec((1, 1, _R2, _F), lambda b, p: (b, p, 0, 0)),
        out_shape=jax.ShapeDtypeStruct((_BT, 3, _R2, _F), jnp.float32),
        scratch_shapes=[pltpu.VMEM((_R2, _F), jnp.float32)],
    )(idx3, c)


def kernel(verts, emb, W_pos, b_pos, W0_0, b0_0, W1_0, b1_0, Ws_0,
           W0_1, b0_1, W1_1, b1_1, Ws_1, W0_2, b0_2, W1_2, b1_2, Ws_2,
           W_c, b_c):
    f32 = jnp.float32
    pad_n = _NP - _N

    verts_p = jnp.pad(verts, ((0, 0), (0, pad_n), (0, 5)))
    emb_p = jnp.pad(emb, ((0, pad_n), (0, 0)))

    pos = jnp.clip(verts, -2.0 + 1e-6, 2.0 - 1e-6)
    q = (pos * _RES).astype(jnp.int32)
    mk = lambda a, b2: jnp.clip(q[..., a] + _RES * q[..., b2], 0, _R2 - 1)
    idx3 = jnp.stack([mk(0, 1), mk(0, 2), mk(1, 2)], axis=1)
    idx3 = jnp.pad(idx3, ((0, 0), (0, 0), (0, pad_n)))
    idx3 = idx3.reshape(_BT * 3, 1, _NP)

    w3 = jnp.pad(W_pos[:3], ((0, 5), (0, 0)))
    we = W_pos[3:]
    row = lambda b: b.reshape(1, -1).astype(f32)

    net1 = _dense1(verts_p, emb_p, w3, we, row(b_pos),
                   W0_0, row(b0_0), W1_0, row(b1_0), Ws_0)
    pooled1 = _pool(net1, idx3)
    net2 = _dense23(net1, pooled1, W0_1, row(b0_1), W1_1, row(b1_1), Ws_1,
                    W_c, row(b_c), project=False)
    pooled2 = _pool(net2, idx3)
    c = _dense23(net2, pooled2, W0_2, row(b0_2), W1_2, row(b1_2), Ws_2,
                 W_c, row(b_c), project=True)
    planes = _mean(c, idx3)
    planes = jnp.swapaxes(planes, 2, 3)
    return planes.reshape(_B, _T, 3, _F, _RES, _RES)
